# baseline (speedup 1.0000x reference)
import jax
import jax.numpy as jnp
from jax import lax
from jax.experimental import pallas as pl
from jax.experimental.pallas import tpu as pltpu
from jax.experimental.pallas import tpu_sc as plsc

_N = 100000
_E = 1600000
_HALF = 16
_NT = 16
_EPT = _E // _NT
_CH = 400
_NCH = _EPT // _CH
_NG = (_NCH - 2) // 4
_RPT = _N // _NT
_ECH = 125
_M = _RPT // _ECH


def _sc_body(w4, e1, e2, pos_o, neg_o, h_o, acc,
             si0, si1, si2, si3, di0, di1, di2, di3,
             rw0, rw1, rw2, rw3,
             smi0, smi1, smi2, smi3, smg0, smg1, smg2, smg3,
             sms0, sms1, sms2, sms3):
    c = lax.axis_index("c")
    s = lax.axis_index("s")
    row0 = s * _RPT
    srci = (si0, si1, si2, si3)
    dsti = (di0, di1, di2, di3)
    rows = (rw0, rw1, rw2, rw3)
    semi = (smi0, smi1, smi2, smi3)
    semg = (smg0, smg1, smg2, smg3)
    sems = (sms0, sms1, sms2, sms3)
    ebuf = rw0
    zb = rw2

    def _acc_chunk(m):
        off = pl.multiple_of(row0 + m * _ECH, 8)
        return acc.at[pl.ds(off, _ECH)]

    def _zb_fill():
        def zset(i, _):
            zb[i, :] = jnp.zeros((_HALF,), jnp.float32)
            return 0
        lax.fori_loop(0, _ECH, zset, 0)

    _zb_fill()

    def _z(m, _):
        pltpu.sync_copy(zb.at[pl.ds(0, _ECH)], _acc_chunk(m))
        return 0
    lax.fori_loop(0, _M, _z, 0)
    plsc.subcore_barrier()

    def _accumulate(e, table, do_xform):
        base = s * _EPT

        def _eoff(k):
            return pl.multiple_of(base + k * _CH, 8)

        def idx_start(k, j):
            off = _eoff(k)
            pltpu.async_copy(e.at[0, pl.ds(off, _CH)], srci[j], semi[j])
            pltpu.async_copy(e.at[1, pl.ds(off, _CH)], dsti[j], semi[j])

        def idx_wait(k, j):
            off = _eoff(k)
            pltpu.make_async_copy(e.at[0, pl.ds(off, _CH)], srci[j],
                                  semi[j]).wait()
            pltpu.make_async_copy(e.at[1, pl.ds(off, _CH)], dsti[j],
                                  semi[j]).wait()

        def xform(j):
            if not do_xform:
                return
            sl = srci[j]

            def body(i, _):
                v = sl[pl.ds(i * 16, 16)]
                sl[pl.ds(i * 16, 16)] = v + v + c
                return 0
            lax.fori_loop(0, _CH // 16, body, 0)

        def gather_start(j):
            pltpu.async_copy(table.at[srci[j]], rows[j], semg[j])

        def gather_wait(j):
            pltpu.make_async_copy(table.at[srci[j]], rows[j], semg[j]).wait()

        def scat_start(j):
            pltpu.async_copy(rows[j], acc.at[dsti[j]], sems[j], add=True)

        def scat_wait(j):
            pltpu.make_async_copy(rows[j], acc.at[dsti[j]], sems[j]).wait()

        idx_start(0, 0)
        idx_start(1, 1)
        idx_wait(0, 0)
        xform(0)
        gather_start(0)

        def group(g, _):
            for j in range(4):
                k = g * 4 + j

                @pl.when(k >= 2)
                def _(j=j):
                    scat_wait((j + 2) % 4)

                @pl.when(k + 2 < _NCH)
                def _(k=k, j=j):
                    idx_start(k + 2, (j + 2) % 4)

                @pl.when(k + 1 < _NCH)
                def _(k=k, j=j):
                    idx_wait(k + 1, (j + 1) % 4)
                    xform((j + 1) % 4)
                    gather_start((j + 1) % 4)
                gather_wait(j)
                scat_start(j)
            return 0
        lax.fori_loop(0, _NG, group, 0)

        scat_wait(2)
        idx_wait(_NCH - 1, 1)
        xform(1)
        gather_start(1)
        gather_wait(0)
        scat_start(0)
        scat_wait(3)
        gather_wait(1)
        scat_start(1)
        scat_wait(0)
        scat_wait(1)
        plsc.subcore_barrier()

    def _evacuate(out_at, do_relu):
        _zb_fill()

        def step(m, _):
            pltpu.sync_copy(_acc_chunk(m), ebuf.at[pl.ds(0, _ECH)])
            if do_relu:
                def relu_row(i, _):
                    ebuf[i, :] = jnp.maximum(ebuf[i, :], 0.0)
                    return 0
                lax.fori_loop(0, _ECH, relu_row, 0)
            pltpu.sync_copy(ebuf.at[pl.ds(0, _ECH)], out_at(m))
            pltpu.sync_copy(zb.at[pl.ds(0, _ECH)], _acc_chunk(m))
            return 0
        lax.fori_loop(0, _M, step, 0)
        plsc.subcore_barrier()

    def _h_at(m):
        off = pl.multiple_of(row0 + m * _ECH, 8)
        return h_o.at[c].at[pl.ds(off, _ECH)]

    for e, out in ((e1, pos_o), (e2, neg_o)):
        def _out_at(m, out=out):
            off = pl.multiple_of(row0 + m * _ECH, 8)
            return out.at[pl.ds(off, _ECH), c]

        _accumulate(e, w4, True)
        _evacuate(_h_at, True)
        _accumulate(e, h_o.at[c], False)
        _evacuate(_out_at, False)


def kernel(inputs, edge_index1, edge_index2, w):
    del inputs
    w4 = w.reshape(2 * _N, _HALF)
    mesh = plsc.VectorSubcoreMesh(core_axis_name="c", subcore_axis_name="s")
    f = pl.kernel(
        _sc_body,
        out_type=[
            jax.ShapeDtypeStruct((_N, 2, _HALF), jnp.float32),
            jax.ShapeDtypeStruct((_N, 2, _HALF), jnp.float32),
            jax.ShapeDtypeStruct((2, _N, _HALF), jnp.float32),
        ],
        mesh=mesh,
        scratch_types=[
            pltpu.VMEM_SHARED((_N, _HALF), jnp.float32),
            pltpu.VMEM((_CH,), jnp.int32),
            pltpu.VMEM((_CH,), jnp.int32),
            pltpu.VMEM((_CH,), jnp.int32),
            pltpu.VMEM((_CH,), jnp.int32),
            pltpu.VMEM((_CH,), jnp.int32),
            pltpu.VMEM((_CH,), jnp.int32),
            pltpu.VMEM((_CH,), jnp.int32),
            pltpu.VMEM((_CH,), jnp.int32),
            pltpu.VMEM((_CH, _HALF), jnp.float32),
            pltpu.VMEM((_CH, _HALF), jnp.float32),
            pltpu.VMEM((_CH, _HALF), jnp.float32),
            pltpu.VMEM((_CH, _HALF), jnp.float32),
            pltpu.SemaphoreType.DMA,
            pltpu.SemaphoreType.DMA,
            pltpu.SemaphoreType.DMA,
            pltpu.SemaphoreType.DMA,
            pltpu.SemaphoreType.DMA,
            pltpu.SemaphoreType.DMA,
            pltpu.SemaphoreType.DMA,
            pltpu.SemaphoreType.DMA,
            pltpu.SemaphoreType.DMA,
            pltpu.SemaphoreType.DMA,
            pltpu.SemaphoreType.DMA,
            pltpu.SemaphoreType.DMA,
        ],
        compiler_params=pltpu.CompilerParams(use_tc_tiling_on_sc=False),
    )
    pos4, neg4, _ = f(w4, edge_index1, edge_index2)
    return pos4.reshape(_N, 32), neg4.reshape(_N, 32)


# speedup vs baseline: 1.0132x; 1.0132x over previous
import jax
import jax.numpy as jnp
from jax import lax
from jax.experimental import pallas as pl
from jax.experimental.pallas import tpu as pltpu
from jax.experimental.pallas import tpu_sc as plsc

_N = 100000
_E = 1600000
_HALF = 16
_NT = 16
_EPT = _E // _NT
_CH = 400
_NCH = _EPT // _CH
_NG = (_NCH - 2) // 4
_RPT = _N // _NT
_ECH = 250
_M = _RPT // _ECH


def _sc_body(w4, e1, e2, pos_o, neg_o, h_o, acc,
             si0, si1, si2, si3, di0, di1, di2, di3,
             rw0, rw1, rw2, rw3,
             smi0, smi1, smi2, smi3, smg0, smg1, smg2, smg3,
             sms0, sms1, sms2, sms3):
    c = lax.axis_index("c")
    s = lax.axis_index("s")
    row0 = s * _RPT
    srci = (si0, si1, si2, si3)
    dsti = (di0, di1, di2, di3)
    rows = (rw0, rw1, rw2, rw3)
    semi = (smi0, smi1, smi2, smi3)
    semg = (smg0, smg1, smg2, smg3)
    sems = (sms0, sms1, sms2, sms3)
    ebuf = rw0
    zb = rw2

    def _acc_chunk(m):
        off = pl.multiple_of(row0 + m * _ECH, 8)
        return acc.at[pl.ds(off, _ECH)]

    def _zb_fill():
        def zset(i, _):
            zb[i, :] = jnp.zeros((_HALF,), jnp.float32)
            return 0
        lax.fori_loop(0, _ECH, zset, 0)

    _zb_fill()

    def _z(m, _):
        pltpu.sync_copy(zb.at[pl.ds(0, _ECH)], _acc_chunk(m))
        return 0
    lax.fori_loop(0, _M, _z, 0)
    plsc.subcore_barrier()

    def _accumulate(e, table, do_xform):
        base = s * _EPT

        def _eoff(k):
            return pl.multiple_of(base + k * _CH, 8)

        def idx_start(k, j):
            off = _eoff(k)
            pltpu.async_copy(e.at[0, pl.ds(off, _CH)], srci[j], semi[j])
            pltpu.async_copy(e.at[1, pl.ds(off, _CH)], dsti[j], semi[j])

        def idx_wait(k, j):
            off = _eoff(k)
            pltpu.make_async_copy(e.at[0, pl.ds(off, _CH)], srci[j],
                                  semi[j]).wait()
            pltpu.make_async_copy(e.at[1, pl.ds(off, _CH)], dsti[j],
                                  semi[j]).wait()

        def xform(j):
            if not do_xform:
                return
            sl = srci[j]

            def body(i, _):
                v = sl[pl.ds(i * 16, 16)]
                sl[pl.ds(i * 16, 16)] = v + v + c
                return 0
            lax.fori_loop(0, _CH // 16, body, 0)

        def gather_start(j):
            pltpu.async_copy(table.at[srci[j]], rows[j], semg[j])

        def gather_wait(j):
            pltpu.make_async_copy(table.at[srci[j]], rows[j], semg[j]).wait()

        def scat_start(j):
            pltpu.async_copy(rows[j], acc.at[dsti[j]], sems[j], add=True)

        def scat_wait(j):
            pltpu.make_async_copy(rows[j], acc.at[dsti[j]], sems[j]).wait()

        idx_start(0, 0)
        idx_start(1, 1)
        idx_wait(0, 0)
        xform(0)
        gather_start(0)

        def group(g, _):
            for j in range(4):
                k = g * 4 + j

                @pl.when(k >= 2)
                def _(j=j):
                    scat_wait((j + 2) % 4)

                @pl.when(k + 2 < _NCH)
                def _(k=k, j=j):
                    idx_start(k + 2, (j + 2) % 4)

                @pl.when(k + 1 < _NCH)
                def _(k=k, j=j):
                    idx_wait(k + 1, (j + 1) % 4)
                    xform((j + 1) % 4)
                    gather_start((j + 1) % 4)
                gather_wait(j)
                scat_start(j)
            return 0
        lax.fori_loop(0, _NG, group, 0)

        scat_wait(2)
        idx_wait(_NCH - 1, 1)
        xform(1)
        gather_start(1)
        gather_wait(0)
        scat_start(0)
        scat_wait(3)
        gather_wait(1)
        scat_start(1)
        scat_wait(0)
        scat_wait(1)
        plsc.subcore_barrier()

    def _evacuate(out_at, do_relu):
        _zb_fill()

        def step(m, _):
            pltpu.sync_copy(_acc_chunk(m), ebuf.at[pl.ds(0, _ECH)])
            if do_relu:
                def relu_row(i, _):
                    ebuf[i, :] = jnp.maximum(ebuf[i, :], 0.0)
                    return 0
                lax.fori_loop(0, _ECH, relu_row, 0)
            pltpu.sync_copy(ebuf.at[pl.ds(0, _ECH)], out_at(m))
            pltpu.sync_copy(zb.at[pl.ds(0, _ECH)], _acc_chunk(m))
            return 0
        lax.fori_loop(0, _M, step, 0)
        plsc.subcore_barrier()

    def _h_at(m):
        off = pl.multiple_of(row0 + m * _ECH, 8)
        return h_o.at[c].at[pl.ds(off, _ECH)]

    for e, out in ((e1, pos_o), (e2, neg_o)):
        def _out_at(m, out=out):
            off = pl.multiple_of(row0 + m * _ECH, 8)
            return out.at[pl.ds(off, _ECH), c]

        _accumulate(e, w4, True)
        _evacuate(_h_at, True)
        _accumulate(e, h_o.at[c], False)
        _evacuate(_out_at, False)


def kernel(inputs, edge_index1, edge_index2, w):
    del inputs
    w4 = w.reshape(2 * _N, _HALF)
    mesh = plsc.VectorSubcoreMesh(core_axis_name="c", subcore_axis_name="s")
    f = pl.kernel(
        _sc_body,
        out_type=[
            jax.ShapeDtypeStruct((_N, 2, _HALF), jnp.float32),
            jax.ShapeDtypeStruct((_N, 2, _HALF), jnp.float32),
            jax.ShapeDtypeStruct((2, _N, _HALF), jnp.float32),
        ],
        mesh=mesh,
        scratch_types=[
            pltpu.VMEM_SHARED((_N, _HALF), jnp.float32),
            pltpu.VMEM((_CH,), jnp.int32),
            pltpu.VMEM((_CH,), jnp.int32),
            pltpu.VMEM((_CH,), jnp.int32),
            pltpu.VMEM((_CH,), jnp.int32),
            pltpu.VMEM((_CH,), jnp.int32),
            pltpu.VMEM((_CH,), jnp.int32),
            pltpu.VMEM((_CH,), jnp.int32),
            pltpu.VMEM((_CH,), jnp.int32),
            pltpu.VMEM((_CH, _HALF), jnp.float32),
            pltpu.VMEM((_CH, _HALF), jnp.float32),
            pltpu.VMEM((_CH, _HALF), jnp.float32),
            pltpu.VMEM((_CH, _HALF), jnp.float32),
            pltpu.SemaphoreType.DMA,
            pltpu.SemaphoreType.DMA,
            pltpu.SemaphoreType.DMA,
            pltpu.SemaphoreType.DMA,
            pltpu.SemaphoreType.DMA,
            pltpu.SemaphoreType.DMA,
            pltpu.SemaphoreType.DMA,
            pltpu.SemaphoreType.DMA,
            pltpu.SemaphoreType.DMA,
            pltpu.SemaphoreType.DMA,
            pltpu.SemaphoreType.DMA,
            pltpu.SemaphoreType.DMA,
        ],
        compiler_params=pltpu.CompilerParams(use_tc_tiling_on_sc=False),
    )
    pos4, neg4, _ = f(w4, edge_index1, edge_index2)
    return pos4.reshape(_N, 32), neg4.reshape(_N, 32)
